# initial kernel scaffold (unmeasured)
import jax
import jax.numpy as jnp
from jax import lax
from jax.experimental import pallas as pl
from jax.experimental.pallas import tpu as pltpu

N_DEV = 8
CHUNKS_PER_DEV = 4
NDIR = 1
NQ = CHUNKS_PER_DEV // NDIR


def kernel(x, w_mat):
    m, k_loc = x.shape
    k2, n = w_mat.shape
    c_rows = m // (N_DEV * CHUNKS_PER_DEV)

    def body(x_ref, w_ref, out_ref, acc, recv, tmp,
             send_sems, recv_sems, copy_sem, credit_sems):
        d = lax.axis_index("i")
        right = (d + 1) % N_DEV
        left = (d - 1) % N_DEV
        dirs = list(range(NDIR))
        peers = [right, left]
        ups = [left, right]

        barrier = pltpu.get_barrier_semaphore()
        for nbr in (left, right):
            pl.semaphore_signal(
                barrier, inc=1,
                device_id=(nbr,), device_id_type=pl.DeviceIdType.MESH,
            )
        pl.semaphore_wait(barrier, 2)

        def rows(q, v, dirn):
            c = (v * NQ + q) * NDIR + dirn
            return pl.ds(c * c_rows, c_rows)

        def partial(q, v, dirn):
            return lax.dot_general(
                x_ref[rows(q, v, dirn), :], w_ref[...],
                (((1,), (0,)), ((), ())),
                preferred_element_type=jnp.float32,
            )

        def mk_remote(src, dst, dirn):
            return pltpu.make_async_remote_copy(
                src_ref=src, dst_ref=dst,
                send_sem=send_sems.at[dirn], recv_sem=recv_sems.at[dirn],
                device_id=(peers[dirn],),
                device_id_type=pl.DeviceIdType.MESH,
            )

        def credit_upstream(dirn):
            pl.semaphore_signal(
                credit_sems.at[dirn], inc=1,
                device_id=(ups[dirn],), device_id_type=pl.DeviceIdType.MESH,
            )

        k_sent = [0] * NDIR

        for q in range(NQ):
            for dirn in dirs:
                acc[dirn] = partial(q, d, dirn)

            for s in range(N_DEV - 1):
                rdmas = []
                for dirn in dirs:
                    if k_sent[dirn] > 0:
                        pl.semaphore_wait(credit_sems.at[dirn], 1)
                    r = mk_remote(acc.at[dirn], recv.at[dirn], dirn)
                    r.start()
                    k_sent[dirn] += 1
                    rdmas.append(r)
                vns = []
                for dirn in dirs:
                    vn = (d - s - 1) % N_DEV if dirn == 0 else (d + s + 1) % N_DEV
                    vns.append(vn)
                    tmp[dirn] = partial(q, vn, dirn)
                for dirn, r in zip(dirs, rdmas):
                    r.wait_recv()
                    r.wait_send()
                    val = recv[dirn] + tmp[dirn]
                    if s == N_DEV - 2:
                        val = jnp.maximum(val, 0.0)
                    acc[dirn] = val
                    credit_upstream(dirn)

            for dirn in dirs:
                vo = (d + 1) % N_DEV if dirn == 0 else (d - 1) % N_DEV
                cp = pltpu.make_async_copy(
                    acc.at[dirn], out_ref.at[rows(q, vo, dirn), :], copy_sem)
                cp.start()
                cp.wait()

            bufs = [acc, recv]
            for t in range(N_DEV - 1):
                rdmas = []
                for dirn in dirs:
                    pl.semaphore_wait(credit_sems.at[dirn], 1)
                    r = mk_remote(bufs[t % 2].at[dirn],
                                  bufs[(t + 1) % 2].at[dirn], dirn)
                    r.start()
                    k_sent[dirn] += 1
                    rdmas.append(r)
                for dirn, r in zip(dirs, rdmas):
                    vr = (d - t) % N_DEV if dirn == 0 else (d + t) % N_DEV
                    r.wait_recv()
                    cp = pltpu.make_async_copy(
                        bufs[(t + 1) % 2].at[dirn],
                        out_ref.at[rows(q, vr, dirn), :], copy_sem)
                    cp.start()
                    cp.wait()
                    r.wait_send()
                    credit_upstream(dirn)

        for dirn in dirs:
            pl.semaphore_wait(credit_sems.at[dirn], 1)

    return pl.pallas_call(
        body,
        out_shape=jax.ShapeDtypeStruct((m, n), jnp.float32),
        in_specs=[
            pl.BlockSpec(memory_space=pltpu.VMEM),
            pl.BlockSpec(memory_space=pltpu.VMEM),
        ],
        out_specs=pl.BlockSpec(memory_space=pltpu.ANY),
        scratch_shapes=[
            pltpu.VMEM((2, c_rows, n), jnp.float32),
            pltpu.VMEM((2, c_rows, n), jnp.float32),
            pltpu.VMEM((2, c_rows, n), jnp.float32),
            pltpu.SemaphoreType.DMA((2,)),
            pltpu.SemaphoreType.DMA((2,)),
            pltpu.SemaphoreType.DMA,
            pltpu.SemaphoreType.REGULAR((2,)),
        ],
        compiler_params=pltpu.CompilerParams(collective_id=0),
    )(x, w_mat)


# baseline (device time: 2868255 ns/iter reference)
import jax
import jax.numpy as jnp
from jax import lax
from jax.experimental import pallas as pl
from jax.experimental.pallas import tpu as pltpu

N_DEV = 8
CHUNKS_PER_DEV = 4
NDIR = 1
NQ = CHUNKS_PER_DEV // NDIR


def kernel(x, w_mat):
    m, k_loc = x.shape
    k2, n = w_mat.shape
    c_rows = m // (N_DEV * CHUNKS_PER_DEV)

    def body(x_ref, w_ref, out_ref, acc, recv, tmp,
             send_sems, recv_sems, copy_sem, credit_sems):
        d = lax.axis_index("i")
        right = (d + 1) % N_DEV
        left = (d - 1) % N_DEV
        dirs = list(range(NDIR))
        peers = [right, left]
        ups = [left, right]

        barrier = pltpu.get_barrier_semaphore()
        for nbr in (left, right):
            pl.semaphore_signal(
                barrier, inc=1,
                device_id=(nbr,), device_id_type=pl.DeviceIdType.MESH,
            )
        pl.semaphore_wait(barrier, 2)

        def rows(q, v, dirn):
            c = (v * NQ + q) * NDIR + dirn
            return pl.ds(c * c_rows, c_rows)

        def partial(q, v, dirn):
            return lax.dot_general(
                x_ref[rows(q, v, dirn), :], w_ref[...],
                (((1,), (0,)), ((), ())),
                preferred_element_type=jnp.float32,
            )

        def mk_remote(src, dst, dirn):
            return pltpu.make_async_remote_copy(
                src_ref=src, dst_ref=dst,
                send_sem=send_sems.at[dirn], recv_sem=recv_sems.at[dirn],
                device_id=(peers[dirn],),
                device_id_type=pl.DeviceIdType.MESH,
            )

        def credit_upstream(dirn):
            pl.semaphore_signal(
                credit_sems.at[dirn], inc=1,
                device_id=(ups[dirn],), device_id_type=pl.DeviceIdType.MESH,
            )

        k_sent = [0] * NDIR

        for q in range(NQ):
            for dirn in dirs:
                acc[dirn] = partial(q, d, dirn)

            for s in range(N_DEV - 1):
                rdmas = []
                for dirn in dirs:
                    if k_sent[dirn] > 0:
                        pl.semaphore_wait(credit_sems.at[dirn], 1)
                    r = mk_remote(acc.at[dirn], recv.at[dirn], dirn)
                    r.start()
                    k_sent[dirn] += 1
                    rdmas.append(r)
                vns = []
                for dirn in dirs:
                    vn = (d - s - 1) % N_DEV if dirn == 0 else (d + s + 1) % N_DEV
                    vns.append(vn)
                    tmp[dirn] = partial(q, vn, dirn)
                for dirn, r in zip(dirs, rdmas):
                    r.wait_recv()
                    r.wait_send()
                    val = recv[dirn] + tmp[dirn]
                    if s == N_DEV - 2:
                        val = jnp.maximum(val, 0.0)
                    acc[dirn] = val
                    credit_upstream(dirn)

            for dirn in dirs:
                vo = (d + 1) % N_DEV if dirn == 0 else (d - 1) % N_DEV
                cp = pltpu.make_async_copy(
                    acc.at[dirn], out_ref.at[rows(q, vo, dirn), :], copy_sem)
                cp.start()
                cp.wait()

            bufs = [acc, recv]
            for t in range(N_DEV - 1):
                rdmas = []
                for dirn in dirs:
                    pl.semaphore_wait(credit_sems.at[dirn], 1)
                    r = mk_remote(bufs[t % 2].at[dirn],
                                  bufs[(t + 1) % 2].at[dirn], dirn)
                    r.start()
                    k_sent[dirn] += 1
                    rdmas.append(r)
                for dirn, r in zip(dirs, rdmas):
                    vr = (d - t) % N_DEV if dirn == 0 else (d + t) % N_DEV
                    r.wait_recv()
                    cp = pltpu.make_async_copy(
                        bufs[(t + 1) % 2].at[dirn],
                        out_ref.at[rows(q, vr, dirn), :], copy_sem)
                    cp.start()
                    cp.wait()
                    r.wait_send()
                    credit_upstream(dirn)

        for dirn in dirs:
            pl.semaphore_wait(credit_sems.at[dirn], 1)

    return pl.pallas_call(
        body,
        out_shape=jax.ShapeDtypeStruct((m, n), jnp.float32),
        in_specs=[
            pl.BlockSpec(memory_space=pltpu.VMEM),
            pl.BlockSpec(memory_space=pltpu.VMEM),
        ],
        out_specs=pl.BlockSpec(memory_space=pl.ANY),
        scratch_shapes=[
            pltpu.VMEM((2, c_rows, n), jnp.float32),
            pltpu.VMEM((2, c_rows, n), jnp.float32),
            pltpu.VMEM((2, c_rows, n), jnp.float32),
            pltpu.SemaphoreType.DMA((2,)),
            pltpu.SemaphoreType.DMA((2,)),
            pltpu.SemaphoreType.DMA,
            pltpu.SemaphoreType.REGULAR((2,)),
        ],
        compiler_params=pltpu.CompilerParams(
            collective_id=0, vmem_limit_bytes=40 * 1024 * 1024),
    )(x, w_mat)


# device time: 1502775 ns/iter; 1.9086x vs baseline; 1.9086x over previous
import jax
import jax.numpy as jnp
from jax import lax
from jax.experimental import pallas as pl
from jax.experimental.pallas import tpu as pltpu

N_DEV = 8
CHUNKS_PER_DEV = 4
NDIR = 2
NQ = CHUNKS_PER_DEV // NDIR


def kernel(x, w_mat):
    m, k_loc = x.shape
    k2, n = w_mat.shape
    c_rows = m // (N_DEV * CHUNKS_PER_DEV)

    def body(x_ref, w_ref, out_ref, acc, recv, tmp,
             send_sems, recv_sems, copy_sem, credit_sems):
        d = lax.axis_index("i")
        right = (d + 1) % N_DEV
        left = (d - 1) % N_DEV
        dirs = list(range(NDIR))
        peers = [right, left]
        ups = [left, right]

        barrier = pltpu.get_barrier_semaphore()
        for nbr in (left, right):
            pl.semaphore_signal(
                barrier, inc=1,
                device_id=(nbr,), device_id_type=pl.DeviceIdType.MESH,
            )
        pl.semaphore_wait(barrier, 2)

        def rows(q, v, dirn):
            c = (v * NQ + q) * NDIR + dirn
            return pl.ds(c * c_rows, c_rows)

        def partial(q, v, dirn):
            return lax.dot_general(
                x_ref[rows(q, v, dirn), :], w_ref[...],
                (((1,), (0,)), ((), ())),
                preferred_element_type=jnp.float32,
            )

        def mk_remote(src, dst, dirn):
            return pltpu.make_async_remote_copy(
                src_ref=src, dst_ref=dst,
                send_sem=send_sems.at[dirn], recv_sem=recv_sems.at[dirn],
                device_id=(peers[dirn],),
                device_id_type=pl.DeviceIdType.MESH,
            )

        def credit_upstream(dirn):
            pl.semaphore_signal(
                credit_sems.at[dirn], inc=1,
                device_id=(ups[dirn],), device_id_type=pl.DeviceIdType.MESH,
            )

        k_sent = [0] * NDIR

        for q in range(NQ):
            for dirn in dirs:
                acc[dirn] = partial(q, d, dirn)

            for s in range(N_DEV - 1):
                rdmas = []
                for dirn in dirs:
                    if k_sent[dirn] > 0:
                        pl.semaphore_wait(credit_sems.at[dirn], 1)
                    r = mk_remote(acc.at[dirn], recv.at[dirn], dirn)
                    r.start()
                    k_sent[dirn] += 1
                    rdmas.append(r)
                vns = []
                for dirn in dirs:
                    vn = (d - s - 1) % N_DEV if dirn == 0 else (d + s + 1) % N_DEV
                    vns.append(vn)
                    tmp[dirn] = partial(q, vn, dirn)
                for dirn, r in zip(dirs, rdmas):
                    r.wait_recv()
                    r.wait_send()
                    val = recv[dirn] + tmp[dirn]
                    if s == N_DEV - 2:
                        val = jnp.maximum(val, 0.0)
                    acc[dirn] = val
                    credit_upstream(dirn)

            for dirn in dirs:
                vo = (d + 1) % N_DEV if dirn == 0 else (d - 1) % N_DEV
                cp = pltpu.make_async_copy(
                    acc.at[dirn], out_ref.at[rows(q, vo, dirn), :], copy_sem)
                cp.start()
                cp.wait()

            bufs = [acc, recv]
            for t in range(N_DEV - 1):
                rdmas = []
                for dirn in dirs:
                    pl.semaphore_wait(credit_sems.at[dirn], 1)
                    r = mk_remote(bufs[t % 2].at[dirn],
                                  bufs[(t + 1) % 2].at[dirn], dirn)
                    r.start()
                    k_sent[dirn] += 1
                    rdmas.append(r)
                for dirn, r in zip(dirs, rdmas):
                    vr = (d - t) % N_DEV if dirn == 0 else (d + t) % N_DEV
                    r.wait_recv()
                    cp = pltpu.make_async_copy(
                        bufs[(t + 1) % 2].at[dirn],
                        out_ref.at[rows(q, vr, dirn), :], copy_sem)
                    cp.start()
                    cp.wait()
                    r.wait_send()
                    credit_upstream(dirn)

        for dirn in dirs:
            pl.semaphore_wait(credit_sems.at[dirn], 1)

    return pl.pallas_call(
        body,
        out_shape=jax.ShapeDtypeStruct((m, n), jnp.float32),
        in_specs=[
            pl.BlockSpec(memory_space=pltpu.VMEM),
            pl.BlockSpec(memory_space=pltpu.VMEM),
        ],
        out_specs=pl.BlockSpec(memory_space=pl.ANY),
        scratch_shapes=[
            pltpu.VMEM((2, c_rows, n), jnp.float32),
            pltpu.VMEM((2, c_rows, n), jnp.float32),
            pltpu.VMEM((2, c_rows, n), jnp.float32),
            pltpu.SemaphoreType.DMA((2,)),
            pltpu.SemaphoreType.DMA((2,)),
            pltpu.SemaphoreType.DMA,
            pltpu.SemaphoreType.REGULAR((2,)),
        ],
        compiler_params=pltpu.CompilerParams(
            collective_id=0, vmem_limit_bytes=40 * 1024 * 1024),
    )(x, w_mat)


# device time: 1450984 ns/iter; 1.9768x vs baseline; 1.0357x over previous
import jax
import jax.numpy as jnp
from jax import lax
from jax.experimental import pallas as pl
from jax.experimental.pallas import tpu as pltpu

N_DEV = 8
CHUNKS_PER_DEV = 4
NDIR = 2
NQ = CHUNKS_PER_DEV // NDIR
DEPTH = 2


def kernel(x, w_mat):
    m, k_loc = x.shape
    k2, n = w_mat.shape
    c_rows = m // (N_DEV * CHUNKS_PER_DEV)

    def body(x_ref, w_ref, out_ref, acc, rbuf, tmp,
             send_sems, recv_sems, cp_sems, own_sems, credit_sems):
        d = lax.axis_index("i")
        right = (d + 1) % N_DEV
        left = (d - 1) % N_DEV
        dirs = list(range(NDIR))
        peers = [right, left]
        ups = [left, right]

        barrier = pltpu.get_barrier_semaphore()
        for nbr in (left, right):
            pl.semaphore_signal(
                barrier, inc=1,
                device_id=(nbr,), device_id_type=pl.DeviceIdType.MESH,
            )
        pl.semaphore_wait(barrier, 2)

        def rows(q, v, dirn):
            c = (v * NQ + q) * NDIR + dirn
            return pl.ds(c * c_rows, c_rows)

        def partial(q, v, dirn):
            return lax.dot_general(
                x_ref[rows(q, v, dirn), :], w_ref[...],
                (((1,), (0,)), ((), ())),
                preferred_element_type=jnp.float32,
            )

        def mk_remote(src, dirn, par):
            return pltpu.make_async_remote_copy(
                src_ref=src, dst_ref=rbuf.at[dirn, par],
                send_sem=send_sems.at[dirn, par],
                recv_sem=recv_sems.at[dirn, par],
                device_id=(peers[dirn],),
                device_id_type=pl.DeviceIdType.MESH,
            )

        def credit_upstream(dirn):
            pl.semaphore_signal(
                credit_sems.at[dirn], inc=1,
                device_id=(ups[dirn],), device_id_type=pl.DeviceIdType.MESH,
            )

        k_sent = [0] * NDIR
        prev = [None] * NDIR

        for q in range(NQ):
            for dirn in dirs:
                acc[dirn] = partial(q, d, dirn)

            for s in range(N_DEV - 1):
                rdmas = []
                for dirn in dirs:
                    k = k_sent[dirn]
                    if k >= DEPTH:
                        pl.semaphore_wait(credit_sems.at[dirn], 1)
                    r = mk_remote(acc.at[dirn], dirn, k % DEPTH)
                    r.start()
                    k_sent[dirn] += 1
                    rdmas.append(r)
                for dirn in dirs:
                    vn = (d - s - 1) % N_DEV if dirn == 0 else (d + s + 1) % N_DEV
                    tmp[dirn] = partial(q, vn, dirn)
                for dirn, r in zip(dirs, rdmas):
                    k = k_sent[dirn] - 1
                    r.wait_recv()
                    r.wait_send()
                    val = rbuf[dirn, k % DEPTH] + tmp[dirn]
                    if s == N_DEV - 2:
                        val = jnp.maximum(val, 0.0)
                    acc[dirn] = val
                    credit_upstream(dirn)

            for dirn in dirs:
                vo = (d + 1) % N_DEV if dirn == 0 else (d - 1) % N_DEV
                cp = pltpu.make_async_copy(
                    acc.at[dirn], out_ref.at[rows(q, vo, dirn), :],
                    own_sems.at[dirn])
                cp.start()

            for t in range(N_DEV - 1):
                rdmas = []
                for dirn in dirs:
                    k = k_sent[dirn]
                    if k >= DEPTH:
                        pl.semaphore_wait(credit_sems.at[dirn], 1)
                    if t == 0:
                        src = acc.at[dirn]
                    else:
                        src = rbuf.at[dirn, (k - 1) % DEPTH]
                    r = mk_remote(src, dirn, k % DEPTH)
                    r.start()
                    k_sent[dirn] += 1
                    rdmas.append(r)
                for dirn, r in zip(dirs, rdmas):
                    k = k_sent[dirn] - 1
                    par = k % DEPTH
                    vr = (d - t) % N_DEV if dirn == 0 else (d + t) % N_DEV
                    r.wait_recv()
                    cp = pltpu.make_async_copy(
                        rbuf.at[dirn, par],
                        out_ref.at[rows(q, vr, dirn), :],
                        cp_sems.at[dirn, par])
                    cp.start()
                    r.wait_send()
                    if t >= 1:
                        pr, pcp = prev[dirn]
                        pcp.wait()
                        credit_upstream(dirn)
                    prev[dirn] = (r, cp)

            for dirn in dirs:
                pr, pcp = prev[dirn]
                pcp.wait()
                credit_upstream(dirn)
                prev[dirn] = None
                pltpu.make_async_copy(
                    acc.at[dirn], acc.at[dirn], own_sems.at[dirn]
                ).wait()

        for dirn in dirs:
            pl.semaphore_wait(credit_sems.at[dirn], DEPTH)

    return pl.pallas_call(
        body,
        out_shape=jax.ShapeDtypeStruct((m, n), jnp.float32),
        in_specs=[
            pl.BlockSpec(memory_space=pltpu.VMEM),
            pl.BlockSpec(memory_space=pltpu.VMEM),
        ],
        out_specs=pl.BlockSpec(memory_space=pl.ANY),
        scratch_shapes=[
            pltpu.VMEM((NDIR, c_rows, n), jnp.float32),
            pltpu.VMEM((NDIR, DEPTH, c_rows, n), jnp.float32),
            pltpu.VMEM((NDIR, c_rows, n), jnp.float32),
            pltpu.SemaphoreType.DMA((NDIR, DEPTH)),
            pltpu.SemaphoreType.DMA((NDIR, DEPTH)),
            pltpu.SemaphoreType.DMA((NDIR, DEPTH)),
            pltpu.SemaphoreType.DMA((NDIR,)),
            pltpu.SemaphoreType.REGULAR((NDIR,)),
        ],
        compiler_params=pltpu.CompilerParams(
            collective_id=0, vmem_limit_bytes=60 * 1024 * 1024),
    )(x, w_mat)


# device time: 1438204 ns/iter; 1.9943x vs baseline; 1.0089x over previous
import jax
import jax.numpy as jnp
from jax import lax
from jax.experimental import pallas as pl
from jax.experimental.pallas import tpu as pltpu

N_DEV = 8
CHUNKS_PER_DEV = 4
NDIR = 2
NH = 2
NQ = CHUNKS_PER_DEV // NDIR
DEPTH = 2


def kernel(x, w_mat):
    m, k_loc = x.shape
    k2, n = w_mat.shape
    c_rows = m // (N_DEV * CHUNKS_PER_DEV)
    hcols = n // NH

    def body(x_ref, w_ref, out_ref, acc, rbuf, tmp,
             send_sems, recv_sems, cp_sems, own_sems, credit_sems):
        d = lax.axis_index("i")
        right = (d + 1) % N_DEV
        left = (d - 1) % N_DEV
        lanes = [(dirn, h) for dirn in range(NDIR) for h in range(NH)]
        peers = [right, left]
        ups = [left, right]

        barrier = pltpu.get_barrier_semaphore()
        for nbr in (left, right):
            pl.semaphore_signal(
                barrier, inc=1,
                device_id=(nbr,), device_id_type=pl.DeviceIdType.MESH,
            )
        pl.semaphore_wait(barrier, 2)

        def rows(q, v, dirn):
            c = (v * NQ + q) * NDIR + dirn
            return pl.ds(c * c_rows, c_rows)

        def cols(h):
            return pl.ds(h * hcols, hcols)

        def partial(q, v, dirn, h):
            return lax.dot_general(
                x_ref[rows(q, v, dirn), :], w_ref[:, cols(h)],
                (((1,), (0,)), ((), ())),
                preferred_element_type=jnp.float32,
            )

        def mk_remote(src, dirn, h, par):
            return pltpu.make_async_remote_copy(
                src_ref=src, dst_ref=rbuf.at[dirn, h, par],
                send_sem=send_sems.at[dirn, h, par],
                recv_sem=recv_sems.at[dirn, h, par],
                device_id=(peers[dirn],),
                device_id_type=pl.DeviceIdType.MESH,
            )

        def credit_upstream(dirn, h):
            pl.semaphore_signal(
                credit_sems.at[dirn, h], inc=1,
                device_id=(ups[dirn],), device_id_type=pl.DeviceIdType.MESH,
            )

        def start_send(dirn, h, src):
            k = k_sent[(dirn, h)]
            if k >= DEPTH:
                pl.semaphore_wait(credit_sems.at[dirn, h], 1)
            r = mk_remote(src, dirn, h, k % DEPTH)
            r.start()
            k_sent[(dirn, h)] = k + 1
            return r

        k_sent = {ln: 0 for ln in lanes}

        for q in range(NQ):
            for dirn, h in lanes:
                acc[dirn, h] = partial(q, d, dirn, h)

            for s in range(N_DEV - 1):
                rdmas = {}
                for dirn, h in lanes:
                    rdmas[(dirn, h)] = start_send(dirn, h, acc.at[dirn, h])
                for dirn, h in lanes:
                    vn = (d - s - 1) % N_DEV if dirn == 0 else (d + s + 1) % N_DEV
                    tmp[dirn, h] = partial(q, vn, dirn, h)
                for dirn, h in lanes:
                    r = rdmas[(dirn, h)]
                    par = (k_sent[(dirn, h)] - 1) % DEPTH
                    r.wait_recv()
                    r.wait_send()
                    val = rbuf[dirn, h, par] + tmp[dirn, h]
                    if s == N_DEV - 2:
                        val = jnp.maximum(val, 0.0)
                    acc[dirn, h] = val
                    credit_upstream(dirn, h)

            for dirn, h in lanes:
                vo = (d + 1) % N_DEV if dirn == 0 else (d - 1) % N_DEV
                pltpu.make_async_copy(
                    acc.at[dirn, h],
                    out_ref.at[rows(q, vo, dirn), cols(h)],
                    own_sems.at[dirn, h]).start()

            cp_prev = {}
            for t in range(N_DEV - 1):
                rdmas = {}
                for dirn, h in lanes:
                    k = k_sent[(dirn, h)]
                    if t == 0:
                        src = acc.at[dirn, h]
                    else:
                        src = rbuf.at[dirn, h, (k - 1) % DEPTH]
                    rdmas[(dirn, h)] = start_send(dirn, h, src)
                for dirn, h in lanes:
                    r = rdmas[(dirn, h)]
                    par = (k_sent[(dirn, h)] - 1) % DEPTH
                    vr = (d - t) % N_DEV if dirn == 0 else (d + t) % N_DEV
                    r.wait_recv()
                    cp = pltpu.make_async_copy(
                        rbuf.at[dirn, h, par],
                        out_ref.at[rows(q, vr, dirn), cols(h)],
                        cp_sems.at[dirn, h, par])
                    cp.start()
                    r.wait_send()
                    if (dirn, h) in cp_prev:
                        cp_prev.pop((dirn, h)).wait()
                        credit_upstream(dirn, h)
                    cp_prev[(dirn, h)] = cp

            for dirn, h in lanes:
                cp_prev.pop((dirn, h)).wait()
                credit_upstream(dirn, h)
                pltpu.make_async_copy(
                    acc.at[dirn, h], acc.at[dirn, h], own_sems.at[dirn, h]
                ).wait()

        for dirn, h in lanes:
            pl.semaphore_wait(credit_sems.at[dirn, h], DEPTH)

    return pl.pallas_call(
        body,
        out_shape=jax.ShapeDtypeStruct((m, n), jnp.float32),
        in_specs=[
            pl.BlockSpec(memory_space=pltpu.VMEM),
            pl.BlockSpec(memory_space=pltpu.VMEM),
        ],
        out_specs=pl.BlockSpec(memory_space=pl.ANY),
        scratch_shapes=[
            pltpu.VMEM((NDIR, NH, c_rows, hcols), jnp.float32),
            pltpu.VMEM((NDIR, NH, DEPTH, c_rows, hcols), jnp.float32),
            pltpu.VMEM((NDIR, NH, c_rows, hcols), jnp.float32),
            pltpu.SemaphoreType.DMA((NDIR, NH, DEPTH)),
            pltpu.SemaphoreType.DMA((NDIR, NH, DEPTH)),
            pltpu.SemaphoreType.DMA((NDIR, NH, DEPTH)),
            pltpu.SemaphoreType.DMA((NDIR, NH)),
            pltpu.SemaphoreType.REGULAR((NDIR, NH)),
        ],
        compiler_params=pltpu.CompilerParams(
            collective_id=0, vmem_limit_bytes=60 * 1024 * 1024),
    )(x, w_mat)


# device time: 1411762 ns/iter; 2.0317x vs baseline; 1.0187x over previous
import jax
import jax.numpy as jnp
from jax import lax
from jax.experimental import pallas as pl
from jax.experimental.pallas import tpu as pltpu

N_DEV = 8
CHUNKS_PER_DEV = 4
NDIR = 2
NH = 2
NQ = CHUNKS_PER_DEV // NDIR
DEPTH = 2


def kernel(x, w_mat):
    m, k_loc = x.shape
    k2, n = w_mat.shape
    c_rows = m // (N_DEV * CHUNKS_PER_DEV)
    hcols = n // NH

    def body(x_ref, w_ref, out_ref, acc, rbuf, tmp,
             send_sems, recv_sems, cp_sems, own_sems, credit_sems):
        d = lax.axis_index("i")
        right = (d + 1) % N_DEV
        left = (d - 1) % N_DEV
        lanes = [(dirn, h) for dirn in range(NDIR) for h in range(NH)]
        peers = [right, left]
        ups = [left, right]

        barrier = pltpu.get_barrier_semaphore()
        for nbr in (left, right):
            pl.semaphore_signal(
                barrier, inc=1,
                device_id=(nbr,), device_id_type=pl.DeviceIdType.MESH,
            )
        pl.semaphore_wait(barrier, 2)

        def rows(q, v, dirn):
            c = (v * NQ + q) * NDIR + dirn
            return pl.ds(c * c_rows, c_rows)

        def cols(h):
            return pl.ds(h * hcols, hcols)

        def partial(q, v, dirn, h):
            return lax.dot_general(
                x_ref[rows(q, v, dirn), :], w_ref[:, cols(h)],
                (((1,), (0,)), ((), ())),
                preferred_element_type=jnp.float32,
            )

        def mk_remote(src, dirn, h, par):
            return pltpu.make_async_remote_copy(
                src_ref=src, dst_ref=rbuf.at[dirn, h, par],
                send_sem=send_sems.at[dirn, h, par],
                recv_sem=recv_sems.at[dirn, h, par],
                device_id=(peers[dirn],),
                device_id_type=pl.DeviceIdType.MESH,
            )

        def credit_upstream(dirn, h):
            pl.semaphore_signal(
                credit_sems.at[dirn, h], inc=1,
                device_id=(ups[dirn],), device_id_type=pl.DeviceIdType.MESH,
            )

        def start_send(dirn, h, src):
            k = k_sent[(dirn, h)]
            if k >= DEPTH:
                pl.semaphore_wait(credit_sems.at[dirn, h], 1)
            r = mk_remote(src, dirn, h, k % DEPTH)
            r.start()
            k_sent[(dirn, h)] = k + 1
            return r

        k_sent = {ln: 0 for ln in lanes}

        for q in range(NQ):
            for dirn, h in lanes:
                acc[dirn, h] = partial(q, d, dirn, h)

            rdmas = {}
            for dirn, h in lanes:
                rdmas[(dirn, h)] = start_send(dirn, h, acc.at[dirn, h])
            for dirn, h in lanes:
                vn = (d - 1) % N_DEV if dirn == 0 else (d + 1) % N_DEV
                tmp[dirn, h] = partial(q, vn, dirn, h)
            for s in range(1, N_DEV - 1):
                for dirn, h in lanes:
                    r = rdmas[(dirn, h)]
                    par = (k_sent[(dirn, h)] - 1) % DEPTH
                    r.wait_recv()
                    r.wait_send()
                    acc[dirn, h] = rbuf[dirn, h, par] + tmp[dirn, h]
                    credit_upstream(dirn, h)
                    rdmas[(dirn, h)] = start_send(dirn, h, acc.at[dirn, h])
                for dirn, h in lanes:
                    vn = (d - s - 1) % N_DEV if dirn == 0 else (d + s + 1) % N_DEV
                    tmp[dirn, h] = partial(q, vn, dirn, h)
            for dirn, h in lanes:
                r = rdmas[(dirn, h)]
                par = (k_sent[(dirn, h)] - 1) % DEPTH
                r.wait_recv()
                r.wait_send()
                acc[dirn, h] = jnp.maximum(
                    rbuf[dirn, h, par] + tmp[dirn, h], 0.0)
                credit_upstream(dirn, h)

            for dirn, h in lanes:
                vo = (d + 1) % N_DEV if dirn == 0 else (d - 1) % N_DEV
                pltpu.make_async_copy(
                    acc.at[dirn, h],
                    out_ref.at[rows(q, vo, dirn), cols(h)],
                    own_sems.at[dirn, h]).start()

            cp_prev = {}
            for t in range(N_DEV - 1):
                rdmas = {}
                for dirn, h in lanes:
                    k = k_sent[(dirn, h)]
                    if t == 0:
                        src = acc.at[dirn, h]
                    else:
                        src = rbuf.at[dirn, h, (k - 1) % DEPTH]
                    rdmas[(dirn, h)] = start_send(dirn, h, src)
                for dirn, h in lanes:
                    r = rdmas[(dirn, h)]
                    par = (k_sent[(dirn, h)] - 1) % DEPTH
                    vr = (d - t) % N_DEV if dirn == 0 else (d + t) % N_DEV
                    r.wait_recv()
                    cp = pltpu.make_async_copy(
                        rbuf.at[dirn, h, par],
                        out_ref.at[rows(q, vr, dirn), cols(h)],
                        cp_sems.at[dirn, h, par])
                    cp.start()
                    r.wait_send()
                    if (dirn, h) in cp_prev:
                        cp_prev.pop((dirn, h)).wait()
                        credit_upstream(dirn, h)
                    cp_prev[(dirn, h)] = cp

            for dirn, h in lanes:
                cp_prev.pop((dirn, h)).wait()
                credit_upstream(dirn, h)
                pltpu.make_async_copy(
                    acc.at[dirn, h], acc.at[dirn, h], own_sems.at[dirn, h]
                ).wait()

        for dirn, h in lanes:
            pl.semaphore_wait(credit_sems.at[dirn, h], DEPTH)

    return pl.pallas_call(
        body,
        out_shape=jax.ShapeDtypeStruct((m, n), jnp.float32),
        in_specs=[
            pl.BlockSpec(memory_space=pltpu.VMEM),
            pl.BlockSpec(memory_space=pltpu.VMEM),
        ],
        out_specs=pl.BlockSpec(memory_space=pl.ANY),
        scratch_shapes=[
            pltpu.VMEM((NDIR, NH, c_rows, hcols), jnp.float32),
            pltpu.VMEM((NDIR, NH, DEPTH, c_rows, hcols), jnp.float32),
            pltpu.VMEM((NDIR, NH, c_rows, hcols), jnp.float32),
            pltpu.SemaphoreType.DMA((NDIR, NH, DEPTH)),
            pltpu.SemaphoreType.DMA((NDIR, NH, DEPTH)),
            pltpu.SemaphoreType.DMA((NDIR, NH, DEPTH)),
            pltpu.SemaphoreType.DMA((NDIR, NH)),
            pltpu.SemaphoreType.REGULAR((NDIR, NH)),
        ],
        compiler_params=pltpu.CompilerParams(
            collective_id=0, vmem_limit_bytes=60 * 1024 * 1024),
    )(x, w_mat)
